# baseline (speedup 1.0000x reference)
import functools
import numpy as np
import jax
import jax.numpy as jnp
from jax import lax
from jax.experimental import pallas as pl
from jax.experimental.pallas import tpu as pltpu
from jax.experimental.pallas import tpu_sc as plsc

_B = 32
_V = 1_000_000
_NS = 10
_CHUNK = 8192
_CC = 16384
_VSC = 18 * _CC
_VTC = _V - _VSC

_ROT_A = (13, 15, 26, 6)
_ROT_B = (17, 29, 16, 24)
_TINY = np.float32(np.finfo(np.float32).tiny)
_LN2 = np.float32(0.6931471805599453)
_SQRT2 = np.float32(1.4142135381698608)


def _np_threefry(k1, k2, x0, x1):
    ks = [np.uint32(k1), np.uint32(k2),
          np.uint32(k1) ^ np.uint32(k2) ^ np.uint32(0x1BD11BDA)]
    x0 = np.uint32((int(x0) + int(ks[0])) & 0xFFFFFFFF)
    x1 = np.uint32((int(x1) + int(ks[1])) & 0xFFFFFFFF)
    rots = (_ROT_A, _ROT_B, _ROT_A, _ROT_B, _ROT_A)
    for i in range(5):
        for r in rots[i]:
            x0 = np.uint32((int(x0) + int(x1)) & 0xFFFFFFFF)
            x1 = np.uint32(((int(x1) << r) | (int(x1) >> (32 - r))) & 0xFFFFFFFF)
            x1 = np.uint32(x1 ^ x0)
        x0 = np.uint32((int(x0) + int(ks[(i + 1) % 3])) & 0xFFFFFFFF)
        x1 = np.uint32((int(x1) + int(ks[(i + 2) % 3]) + i + 1) & 0xFFFFFFFF)
    return x0, x1


_K1, _K2 = _np_threefry(0, 0, 0, 1)
_K3 = np.uint32(_K1 ^ _K2 ^ np.uint32(0x1BD11BDA))


def _threefry_bits(x1):
    ks = (jnp.uint32(_K1), jnp.uint32(_K2), jnp.uint32(_K3))
    x0 = jnp.full_like(x1, _K1)
    x1 = x1 + jnp.uint32(_K2)

    def rnd(a, b, r):
        a = a + b
        b = ((b << jnp.uint32(r)) | (b >> jnp.uint32(32 - r))) ^ a
        return a, b

    rots = (_ROT_A, _ROT_B, _ROT_A, _ROT_B, _ROT_A)
    for i in range(5):
        for r in rots[i]:
            x0, x1 = rnd(x0, x1, r)
        x0 = x0 + ks[(i + 1) % 3]
        x1 = x1 + ks[(i + 2) % 3] + jnp.uint32(i + 1)
    return x0 ^ x1


def _bits_to_u(bits):
    fb = (bits >> jnp.uint32(9)) | jnp.uint32(0x3F800000)
    fl = jax.lax.bitcast_convert_type(fb, jnp.float32) - jnp.float32(1.0)
    return jnp.maximum(_TINY, fl)


def _make_tc(B, V, vhi, K, NS, interpret=False):
    nchunk = (vhi + K - 1) // K
    nsub = K // 128

    def body(l_ref, outf_ref, outi_ref, m_ref, s_ref, bval_ref, bidx_ref):
        j = pl.program_id(0)

        @pl.when(j == 0)
        def _init():
            m_ref[...] = jnp.full_like(m_ref, -jnp.inf)
            s_ref[...] = jnp.zeros_like(s_ref)
            bval_ref[...] = jnp.full_like(bval_ref, -jnp.inf)
            bidx_ref[...] = jnp.zeros_like(bidx_ref)

        lane = jax.lax.broadcasted_iota(jnp.int32, (B, 128), 1)
        rowbase = (jax.lax.broadcasted_iota(jnp.int32, (B, 128), 0) * V).astype(jnp.uint32)
        base_col = j * K

        def sub(t, _):
            col = lane + (base_col + t * 128)
            l = l_ref[:, pl.ds(t * 128, 128)]
            l = jnp.where(col < vhi, l, jnp.float32(-jnp.inf))
            m_ref[...] = jnp.maximum(m_ref[...], l)
            s_ref[...] = s_ref[...] + jnp.exp(l)
            off = rowbase + col.astype(jnp.uint32)
            for s in range(NS):
                u = _bits_to_u(_threefry_bits(off + jnp.uint32(s * B * V)))
                g = -jnp.log(-jnp.log(u))
                phi = g + l
                bv = bval_ref[pl.ds(s * B, B), :]
                better = phi > bv
                bval_ref[pl.ds(s * B, B), :] = jnp.where(better, phi, bv)
                bidx_ref[pl.ds(s * B, B), :] = jnp.where(
                    better, col, bidx_ref[pl.ds(s * B, B), :])
            return 0

        jax.lax.fori_loop(0, nsub, sub, 0, unroll=2)

        @pl.when(j == nchunk - 1)
        def _fin():
            outf_ref[...] = jnp.zeros_like(outf_ref)
            outi_ref[...] = jnp.zeros_like(outi_ref)
            for s in range(NS):
                bv = bval_ref[pl.ds(s * B, B), :]
                vmax = jnp.max(bv, axis=1, keepdims=True)
                cand = jnp.where(bv == vmax, bidx_ref[pl.ds(s * B, B), :],
                                 jnp.int32(0x7FFFFFFF))
                outf_ref[:, s : s + 1] = vmax
                outi_ref[:, s : s + 1] = jnp.min(cand, axis=1, keepdims=True)
            outf_ref[:, NS : NS + 1] = jnp.max(m_ref[...], axis=1, keepdims=True)
            outf_ref[:, NS + 1 : NS + 2] = jnp.sum(s_ref[...], axis=1, keepdims=True)

    return pl.pallas_call(
        body,
        grid=(nchunk,),
        in_specs=[pl.BlockSpec((B, K), lambda j: (0, j))],
        out_specs=[pl.BlockSpec((B, 128), lambda j: (0, 0)),
                   pl.BlockSpec((B, 128), lambda j: (0, 0))],
        out_shape=[jax.ShapeDtypeStruct((B, 128), jnp.float32),
                   jax.ShapeDtypeStruct((B, 128), jnp.int32)],
        scratch_shapes=[
            pltpu.VMEM((B, 128), jnp.float32),
            pltpu.VMEM((B, 128), jnp.float32),
            pltpu.VMEM((NS * B, 128), jnp.float32),
            pltpu.VMEM((NS * B, 128), jnp.int32),
        ],
        interpret=interpret,
    )


def _ln(x):
    bits = jax.lax.bitcast_convert_type(x, jnp.int32)
    e = (bits >> 23) - jnp.int32(127)
    mb = (bits & jnp.int32(0x007FFFFF)) | jnp.int32(0x3F800000)
    m = jax.lax.bitcast_convert_type(mb, jnp.float32)
    big = m >= _SQRT2
    m = jnp.where(big, m * jnp.float32(0.5), m)
    e = jnp.where(big, e + jnp.int32(1), e)
    ef = e.astype(jnp.float32)
    t = (m - jnp.float32(1.0)) / (m + jnp.float32(1.0))
    t2 = t * t
    p = jnp.float32(1.0 / 9.0)
    p = p * t2 + jnp.float32(1.0 / 7.0)
    p = p * t2 + jnp.float32(1.0 / 5.0)
    p = p * t2 + jnp.float32(1.0 / 3.0)
    p = p * t2 + jnp.float32(1.0)
    return ef * _LN2 + (t + t) * p


def _make_sc(B, V, vlo, vsc, CC, NS):
    nch = vsc // CC
    nvec = CC // 16
    mesh = plsc.VectorSubcoreMesh(core_axis_name="c", subcore_axis_name="s")

    @functools.partial(
        pl.kernel,
        mesh=mesh,
        out_type=(jax.ShapeDtypeStruct((B, 208), jnp.float32),
                  jax.ShapeDtypeStruct((B, 160), jnp.int32)),
        scratch_types=[pltpu.VMEM((CC,), jnp.float32),
                       pltpu.VMEM((208,), jnp.float32),
                       pltpu.VMEM((160,), jnp.int32),
                       pltpu.VMEM((NS * 16,), jnp.float32),
                       pltpu.VMEM((NS * 16,), jnp.int32)],
    )
    def sc_kern(l_hbm, outf_hbm, outi_hbm, buf, obf, obi, bvr, bcr):
        w = lax.axis_index("s") * 2 + lax.axis_index("c")
        lane = lax.iota(jnp.int32, 16)

        m0 = jnp.full((16,), -3e38, jnp.float32)
        s0 = jnp.zeros((16,), jnp.float32)
        for s in range(NS):
            bvr[pl.ds(s * 16, 16)] = m0
            bcr[pl.ds(s * 16, 16)] = jnp.zeros((16,), jnp.int32)

        def chunk(c, carry):
            pltpu.sync_copy(l_hbm.at[pl.ds(w * vsc + c * CC, CC)], buf)
            base = vlo + c * CC

            def vec(t, carry):
                m, ssum = carry
                l = buf[pl.ds(t * 16, 16)]
                col = lane + (base + t * 16)
                m = jnp.maximum(m, l)
                ssum = ssum + jnp.exp(l)
                offu = jax.lax.bitcast_convert_type(col + w * V, jnp.uint32)
                for s in range(NS):
                    u = _bits_to_u(_threefry_bits(offu + jnp.uint32(s * B * V)))
                    g = -_ln(-_ln(u))
                    phi = g + l
                    bv = bvr[pl.ds(s * 16, 16)]
                    bt = phi > bv
                    bvr[pl.ds(s * 16, 16)] = jnp.where(bt, phi, bv)
                    bcr[pl.ds(s * 16, 16)] = jnp.where(bt, col, bcr[pl.ds(s * 16, 16)])
                return (m, ssum)

            return lax.fori_loop(0, nvec, vec, carry, unroll=2)

        m, ssum = lax.fori_loop(0, nch, chunk, (m0, s0))

        zf = jnp.zeros((16,), jnp.float32)
        for s in range(NS):
            obf[pl.ds(s * 16, 16)] = bvr[pl.ds(s * 16, 16)]
            obi[pl.ds(s * 16, 16)] = bcr[pl.ds(s * 16, 16)]
        obf[pl.ds(NS * 16, 16)] = m
        obf[pl.ds(NS * 16 + 16, 16)] = ssum
        obf[pl.ds(NS * 16 + 32, 16)] = zf
        pltpu.sync_copy(obf, outf_hbm.at[w])
        pltpu.sync_copy(obi, outi_hbm.at[w])

    return sc_kern


def _make_merge(B, NS, interpret=False):
    def body(tf_ref, ti_ref, sf_ref, si_ref, out_ref):
        intmax = jnp.int32(0x7FFFFFFF)
        m_row = jnp.maximum(
            tf_ref[:, NS : NS + 1],
            jnp.max(sf_ref[:, NS * 16 : NS * 16 + 16], axis=1, keepdims=True))
        s_row = tf_ref[:, NS + 1 : NS + 2] + jnp.sum(
            sf_ref[:, NS * 16 + 16 : NS * 16 + 32], axis=1, keepdims=True)
        top_prob = jnp.exp(m_row) / s_row
        idxs = []
        for s in range(NS):
            scv = sf_ref[:, s * 16 : (s + 1) * 16]
            smax = jnp.max(scv, axis=1, keepdims=True)
            scand = jnp.where(scv == smax, si_ref[:, s * 16 : (s + 1) * 16], intmax)
            sidx = jnp.min(scand, axis=1, keepdims=True)
            use_sc = smax > tf_ref[:, s : s + 1]
            idxs.append(jnp.where(use_sc, sidx, ti_ref[:, s : s + 1]))
        agree_f = jnp.ones_like(top_prob)
        for t in range(1, NS):
            agree_f = agree_f + (idxs[t] == idxs[0]).astype(jnp.float32)
        agree_f = agree_f * jnp.float32(1.0 / NS)
        uniq = jnp.ones_like(top_prob)
        for t in range(1, NS):
            seen = idxs[t] == idxs[0]
            for t2 in range(1, t):
                seen = jnp.logical_or(seen, idxs[t] == idxs[t2])
            uniq = uniq + jnp.float32(1.0) - seen.astype(jnp.float32)
        out_ref[...] = jnp.zeros_like(out_ref)
        out_ref[:, 0:1] = agree_f
        out_ref[:, 1:2] = uniq * jnp.float32(1.0 / NS)
        out_ref[:, 2:3] = agree_f - top_prob

    return pl.pallas_call(
        body,
        out_shape=jax.ShapeDtypeStruct((B, 128), jnp.float32),
        interpret=interpret,
    )


def kernel(logits):
    lsc = logits[:, _VTC:].reshape(_B * _VSC)
    tcf, tci = _make_tc(_B, _V, _VTC, _CHUNK, _NS)(logits)
    scf, sci = _make_sc(_B, _V, _VTC, _VSC, _CC, _NS)(lsc)
    out = _make_merge(_B, _NS)(tcf, tci, scf, sci)
    return out[:, :3]


# speedup vs baseline: 1.3942x; 1.3942x over previous
import functools
import numpy as np
import jax
import jax.numpy as jnp
from jax import lax
from jax.experimental import pallas as pl
from jax.experimental.pallas import tpu as pltpu
from jax.experimental.pallas import tpu_sc as plsc

_B = 32
_V = 1_000_000
_NS = 10
_CHUNK = 8192
_CC = 16384
_VSC = 18 * _CC
_VTC = _V - _VSC

_ROT_A = (13, 15, 26, 6)
_ROT_B = (17, 29, 16, 24)
_TINY = np.float32(np.finfo(np.float32).tiny)
_LN2 = np.float32(0.6931471805599453)
_SQRT2 = np.float32(1.4142135381698608)


def _np_threefry(k1, k2, x0, x1):
    ks = [np.uint32(k1), np.uint32(k2),
          np.uint32(k1) ^ np.uint32(k2) ^ np.uint32(0x1BD11BDA)]
    x0 = np.uint32((int(x0) + int(ks[0])) & 0xFFFFFFFF)
    x1 = np.uint32((int(x1) + int(ks[1])) & 0xFFFFFFFF)
    rots = (_ROT_A, _ROT_B, _ROT_A, _ROT_B, _ROT_A)
    for i in range(5):
        for r in rots[i]:
            x0 = np.uint32((int(x0) + int(x1)) & 0xFFFFFFFF)
            x1 = np.uint32(((int(x1) << r) | (int(x1) >> (32 - r))) & 0xFFFFFFFF)
            x1 = np.uint32(x1 ^ x0)
        x0 = np.uint32((int(x0) + int(ks[(i + 1) % 3])) & 0xFFFFFFFF)
        x1 = np.uint32((int(x1) + int(ks[(i + 2) % 3]) + i + 1) & 0xFFFFFFFF)
    return x0, x1


_K1, _K2 = _np_threefry(0, 0, 0, 1)
_K3 = np.uint32(_K1 ^ _K2 ^ np.uint32(0x1BD11BDA))


def _threefry_bits(x1):
    ks = (jnp.uint32(_K1), jnp.uint32(_K2), jnp.uint32(_K3))
    x0 = jnp.full_like(x1, _K1)
    x1 = x1 + jnp.uint32(_K2)

    def rnd(a, b, r):
        a = a + b
        b = ((b << jnp.uint32(r)) | (b >> jnp.uint32(32 - r))) ^ a
        return a, b

    rots = (_ROT_A, _ROT_B, _ROT_A, _ROT_B, _ROT_A)
    for i in range(5):
        for r in rots[i]:
            x0, x1 = rnd(x0, x1, r)
        x0 = x0 + ks[(i + 1) % 3]
        x1 = x1 + ks[(i + 2) % 3] + jnp.uint32(i + 1)
    return x0 ^ x1


def _bits_to_u(bits):
    fb = (bits >> jnp.uint32(9)) | jnp.uint32(0x3F800000)
    fl = jax.lax.bitcast_convert_type(fb, jnp.float32) - jnp.float32(1.0)
    return jnp.maximum(_TINY, fl)


def _make_tc(B, V, vhi, K, NS, interpret=False):
    nchunk = (vhi + K - 1) // K
    nsub = K // 128

    def body(l_ref, outf_ref, outi_ref, m_ref, s_ref, bval_ref, bidx_ref):
        j = pl.program_id(0)

        @pl.when(j == 0)
        def _init():
            m_ref[...] = jnp.full_like(m_ref, -jnp.inf)
            s_ref[...] = jnp.zeros_like(s_ref)
            bval_ref[...] = jnp.full_like(bval_ref, -jnp.inf)
            bidx_ref[...] = jnp.zeros_like(bidx_ref)

        lane = jax.lax.broadcasted_iota(jnp.int32, (B, 128), 1)
        rowbase = (jax.lax.broadcasted_iota(jnp.int32, (B, 128), 0) * V).astype(jnp.uint32)
        base_col = j * K

        def sub(t, _):
            col = lane + (base_col + t * 128)
            l = l_ref[:, pl.ds(t * 128, 128)]
            l = jnp.where(col < vhi, l, jnp.float32(-jnp.inf))
            m_ref[...] = jnp.maximum(m_ref[...], l)
            s_ref[...] = s_ref[...] + jnp.exp(l)
            off = rowbase + col.astype(jnp.uint32)
            for s in range(NS):
                u = _bits_to_u(_threefry_bits(off + jnp.uint32(s * B * V)))
                g = -jnp.log(-jnp.log(u))
                phi = g + l
                bv = bval_ref[pl.ds(s * B, B), :]
                better = phi > bv
                bval_ref[pl.ds(s * B, B), :] = jnp.where(better, phi, bv)
                bidx_ref[pl.ds(s * B, B), :] = jnp.where(
                    better, col, bidx_ref[pl.ds(s * B, B), :])
            return 0

        jax.lax.fori_loop(0, nsub, sub, 0, unroll=2)

        @pl.when(j == nchunk - 1)
        def _fin():
            outf_ref[...] = jnp.zeros_like(outf_ref)
            outi_ref[...] = jnp.zeros_like(outi_ref)
            for s in range(NS):
                bv = bval_ref[pl.ds(s * B, B), :]
                vmax = jnp.max(bv, axis=1, keepdims=True)
                cand = jnp.where(bv == vmax, bidx_ref[pl.ds(s * B, B), :],
                                 jnp.int32(0x7FFFFFFF))
                outf_ref[:, s : s + 1] = vmax
                outi_ref[:, s : s + 1] = jnp.min(cand, axis=1, keepdims=True)
            outf_ref[:, NS : NS + 1] = jnp.max(m_ref[...], axis=1, keepdims=True)
            outf_ref[:, NS + 1 : NS + 2] = jnp.sum(s_ref[...], axis=1, keepdims=True)

    return pl.pallas_call(
        body,
        grid=(nchunk,),
        in_specs=[pl.BlockSpec((B, K), lambda j: (0, j))],
        out_specs=[pl.BlockSpec((B, 128), lambda j: (0, 0)),
                   pl.BlockSpec((B, 128), lambda j: (0, 0))],
        out_shape=[jax.ShapeDtypeStruct((B, 128), jnp.float32),
                   jax.ShapeDtypeStruct((B, 128), jnp.int32)],
        scratch_shapes=[
            pltpu.VMEM((B, 128), jnp.float32),
            pltpu.VMEM((B, 128), jnp.float32),
            pltpu.VMEM((NS * B, 128), jnp.float32),
            pltpu.VMEM((NS * B, 128), jnp.int32),
        ],
        interpret=interpret,
    )


def _ln(x):
    bits = jax.lax.bitcast_convert_type(x, jnp.int32)
    e = (bits >> 23) - jnp.int32(127)
    mb = (bits & jnp.int32(0x007FFFFF)) | jnp.int32(0x3F800000)
    m = jax.lax.bitcast_convert_type(mb, jnp.float32)
    big = m >= _SQRT2
    m = jnp.where(big, m * jnp.float32(0.5), m)
    e = jnp.where(big, e + jnp.int32(1), e)
    ef = e.astype(jnp.float32)
    t = (m - jnp.float32(1.0)) / (m + jnp.float32(1.0))
    t2 = t * t
    p = jnp.float32(1.0 / 7.0)
    p = p * t2 + jnp.float32(1.0 / 5.0)
    p = p * t2 + jnp.float32(1.0 / 3.0)
    p = p * t2 + jnp.float32(1.0)
    return ef * _LN2 + (t + t) * p


def _make_sc(B, V, vlo, vsc, CC, NS):
    nch = vsc // CC
    nvec = CC // 16
    mesh = plsc.VectorSubcoreMesh(core_axis_name="c", subcore_axis_name="s")

    @functools.partial(
        pl.kernel,
        mesh=mesh,
        out_type=(jax.ShapeDtypeStruct((B, 208), jnp.float32),
                  jax.ShapeDtypeStruct((B, 160), jnp.int32)),
        scratch_types=[pltpu.VMEM((CC,), jnp.float32),
                       pltpu.VMEM((208,), jnp.float32),
                       pltpu.VMEM((160,), jnp.int32)],
    )
    def sc_kern(l_hbm, outf_hbm, outi_hbm, buf, obf, obi):
        w = lax.axis_index("s") * 2 + lax.axis_index("c")
        lane = lax.iota(jnp.int32, 16)

        m0 = jnp.full((16,), -3e38, jnp.float32)
        s0 = jnp.zeros((16,), jnp.float32)
        q0 = tuple(jnp.full((16,), 3e38, jnp.float32) for _ in range(NS))
        bc0 = tuple(jnp.zeros((16,), jnp.int32) for _ in range(NS))

        def chunk(c, carry):
            pltpu.sync_copy(l_hbm.at[pl.ds(w * vsc + c * CC, CC)], buf)
            base = vlo + c * CC

            def vec(t, carry):
                m, ssum, qs, bcs = carry
                l = buf[pl.ds(t * 16, 16)]
                col = lane + (base + t * 16)
                m = jnp.maximum(m, l)
                ssum = ssum + jnp.exp(l)
                wneg = jnp.exp(-l)
                offu = jax.lax.bitcast_convert_type(col + w * V, jnp.uint32)
                nq = []
                nbc = []
                for s in range(NS):
                    u = _bits_to_u(_threefry_bits(offu + jnp.uint32(s * B * V)))
                    q = -_ln(u) * wneg
                    bt = q < qs[s]
                    nq.append(jnp.where(bt, q, qs[s]))
                    nbc.append(jnp.where(bt, col, bcs[s]))
                return (m, ssum, tuple(nq), tuple(nbc))

            return lax.fori_loop(0, nvec, vec, carry, unroll=2)

        m, ssum, qs, bcs = lax.fori_loop(0, nch, chunk, (m0, s0, q0, bc0))

        zf = jnp.zeros((16,), jnp.float32)
        for s in range(NS):
            obf[pl.ds(s * 16, 16)] = qs[s]
            obi[pl.ds(s * 16, 16)] = bcs[s]
        obf[pl.ds(NS * 16, 16)] = m
        obf[pl.ds(NS * 16 + 16, 16)] = ssum
        obf[pl.ds(NS * 16 + 32, 16)] = zf
        pltpu.sync_copy(obf, outf_hbm.at[w])
        pltpu.sync_copy(obi, outi_hbm.at[w])

    return sc_kern


def _make_merge(B, NS, interpret=False):
    def body(tf_ref, ti_ref, sf_ref, si_ref, out_ref):
        intmax = jnp.int32(0x7FFFFFFF)
        m_row = jnp.maximum(
            tf_ref[:, NS : NS + 1],
            jnp.max(sf_ref[:, NS * 16 : NS * 16 + 16], axis=1, keepdims=True))
        s_row = tf_ref[:, NS + 1 : NS + 2] + jnp.sum(
            sf_ref[:, NS * 16 + 16 : NS * 16 + 32], axis=1, keepdims=True)
        top_prob = jnp.exp(m_row) / s_row
        idxs = []
        for s in range(NS):
            sq = sf_ref[:, s * 16 : (s + 1) * 16]
            sminq = jnp.min(sq, axis=1, keepdims=True)
            scand = jnp.where(sq == sminq, si_ref[:, s * 16 : (s + 1) * 16], intmax)
            sidx = jnp.min(scand, axis=1, keepdims=True)
            use_sc = -jnp.log(sminq) > tf_ref[:, s : s + 1]
            idxs.append(jnp.where(use_sc, sidx, ti_ref[:, s : s + 1]))
        agree_f = jnp.ones_like(top_prob)
        for t in range(1, NS):
            agree_f = agree_f + (idxs[t] == idxs[0]).astype(jnp.float32)
        agree_f = agree_f * jnp.float32(1.0 / NS)
        uniq = jnp.ones_like(top_prob)
        for t in range(1, NS):
            seen = idxs[t] == idxs[0]
            for t2 in range(1, t):
                seen = jnp.logical_or(seen, idxs[t] == idxs[t2])
            uniq = uniq + jnp.float32(1.0) - seen.astype(jnp.float32)
        out_ref[...] = jnp.zeros_like(out_ref)
        out_ref[:, 0:1] = agree_f
        out_ref[:, 1:2] = uniq * jnp.float32(1.0 / NS)
        out_ref[:, 2:3] = agree_f - top_prob

    return pl.pallas_call(
        body,
        out_shape=jax.ShapeDtypeStruct((B, 128), jnp.float32),
        interpret=interpret,
    )


def kernel(logits):
    lsc = logits[:, _VTC:].reshape(_B * _VSC)
    tcf, tci = _make_tc(_B, _V, _VTC, _CHUNK, _NS)(logits)
    scf, sci = _make_sc(_B, _V, _VTC, _VSC, _CC, _NS)(lsc)
    out = _make_merge(_B, _NS)(tcf, tci, scf, sci)
    return out[:, :3]


# speedup vs baseline: 1.7246x; 1.2370x over previous
import functools
import numpy as np
import jax
import jax.numpy as jnp
from jax import lax
from jax.experimental import pallas as pl
from jax.experimental.pallas import tpu as pltpu
from jax.experimental.pallas import tpu_sc as plsc

_B = 32
_V = 1_000_000
_NS = 10
_CHUNK = 8192
_CC = 8192
_VSC = 29 * _CC
_VTC = _V - _VSC

_ROT_A = (13, 15, 26, 6)
_ROT_B = (17, 29, 16, 24)
_TINY = np.float32(np.finfo(np.float32).tiny)
_LN2 = np.float32(0.6931471805599453)
_SQRT2 = np.float32(1.4142135381698608)


def _np_threefry(k1, k2, x0, x1):
    ks = [np.uint32(k1), np.uint32(k2),
          np.uint32(k1) ^ np.uint32(k2) ^ np.uint32(0x1BD11BDA)]
    x0 = np.uint32((int(x0) + int(ks[0])) & 0xFFFFFFFF)
    x1 = np.uint32((int(x1) + int(ks[1])) & 0xFFFFFFFF)
    rots = (_ROT_A, _ROT_B, _ROT_A, _ROT_B, _ROT_A)
    for i in range(5):
        for r in rots[i]:
            x0 = np.uint32((int(x0) + int(x1)) & 0xFFFFFFFF)
            x1 = np.uint32(((int(x1) << r) | (int(x1) >> (32 - r))) & 0xFFFFFFFF)
            x1 = np.uint32(x1 ^ x0)
        x0 = np.uint32((int(x0) + int(ks[(i + 1) % 3])) & 0xFFFFFFFF)
        x1 = np.uint32((int(x1) + int(ks[(i + 2) % 3]) + i + 1) & 0xFFFFFFFF)
    return x0, x1


_K1, _K2 = _np_threefry(0, 0, 0, 1)
_K3 = np.uint32(_K1 ^ _K2 ^ np.uint32(0x1BD11BDA))


def _threefry_bits(x1):
    ks = (jnp.uint32(_K1), jnp.uint32(_K2), jnp.uint32(_K3))
    x0 = jnp.full_like(x1, _K1)
    x1 = x1 + jnp.uint32(_K2)

    def rnd(a, b, r):
        a = a + b
        b = ((b << jnp.uint32(r)) | (b >> jnp.uint32(32 - r))) ^ a
        return a, b

    rots = (_ROT_A, _ROT_B, _ROT_A, _ROT_B, _ROT_A)
    for i in range(5):
        for r in rots[i]:
            x0, x1 = rnd(x0, x1, r)
        x0 = x0 + ks[(i + 1) % 3]
        x1 = x1 + ks[(i + 2) % 3] + jnp.uint32(i + 1)
    return x0 ^ x1


def _bits_to_u(bits):
    fb = (bits >> jnp.uint32(9)) | jnp.uint32(0x3F800000)
    fl = jax.lax.bitcast_convert_type(fb, jnp.float32) - jnp.float32(1.0)
    return jnp.maximum(_TINY, fl)


def _make_tc(B, V, vhi, K, NS, interpret=False):
    nchunk = (vhi + K - 1) // K
    nsub = K // 128

    def body(l_ref, outf_ref, outi_ref, m_ref, s_ref, bval_ref, bidx_ref):
        j = pl.program_id(0)

        @pl.when(j == 0)
        def _init():
            m_ref[...] = jnp.full_like(m_ref, -jnp.inf)
            s_ref[...] = jnp.zeros_like(s_ref)
            bval_ref[...] = jnp.full_like(bval_ref, -jnp.inf)
            bidx_ref[...] = jnp.zeros_like(bidx_ref)

        lane = jax.lax.broadcasted_iota(jnp.int32, (B, 128), 1)
        rowbase = (jax.lax.broadcasted_iota(jnp.int32, (B, 128), 0) * V).astype(jnp.uint32)
        base_col = j * K

        def sub(t, _):
            col = lane + (base_col + t * 128)
            l = l_ref[:, pl.ds(t * 128, 128)]
            l = jnp.where(col < vhi, l, jnp.float32(-jnp.inf))
            m_ref[...] = jnp.maximum(m_ref[...], l)
            s_ref[...] = s_ref[...] + jnp.exp(l)
            off = rowbase + col.astype(jnp.uint32)
            for s in range(NS):
                u = _bits_to_u(_threefry_bits(off + jnp.uint32(s * B * V)))
                g = -jnp.log(-jnp.log(u))
                phi = g + l
                bv = bval_ref[pl.ds(s * B, B), :]
                better = phi > bv
                bval_ref[pl.ds(s * B, B), :] = jnp.where(better, phi, bv)
                bidx_ref[pl.ds(s * B, B), :] = jnp.where(
                    better, col, bidx_ref[pl.ds(s * B, B), :])
            return 0

        jax.lax.fori_loop(0, nsub, sub, 0, unroll=4)

        @pl.when(j == nchunk - 1)
        def _fin():
            outf_ref[...] = jnp.zeros_like(outf_ref)
            outi_ref[...] = jnp.zeros_like(outi_ref)
            for s in range(NS):
                bv = bval_ref[pl.ds(s * B, B), :]
                vmax = jnp.max(bv, axis=1, keepdims=True)
                cand = jnp.where(bv == vmax, bidx_ref[pl.ds(s * B, B), :],
                                 jnp.int32(0x7FFFFFFF))
                outf_ref[:, s : s + 1] = vmax
                outi_ref[:, s : s + 1] = jnp.min(cand, axis=1, keepdims=True)
            outf_ref[:, NS : NS + 1] = jnp.max(m_ref[...], axis=1, keepdims=True)
            outf_ref[:, NS + 1 : NS + 2] = jnp.sum(s_ref[...], axis=1, keepdims=True)

    return pl.pallas_call(
        body,
        grid=(nchunk,),
        in_specs=[pl.BlockSpec((B, K), lambda j: (0, j))],
        out_specs=[pl.BlockSpec((B, 128), lambda j: (0, 0)),
                   pl.BlockSpec((B, 128), lambda j: (0, 0))],
        out_shape=[jax.ShapeDtypeStruct((B, 128), jnp.float32),
                   jax.ShapeDtypeStruct((B, 128), jnp.int32)],
        scratch_shapes=[
            pltpu.VMEM((B, 128), jnp.float32),
            pltpu.VMEM((B, 128), jnp.float32),
            pltpu.VMEM((NS * B, 128), jnp.float32),
            pltpu.VMEM((NS * B, 128), jnp.int32),
        ],
        interpret=interpret,
    )


def _ln(x):
    bits = jax.lax.bitcast_convert_type(x, jnp.int32)
    e = (bits >> 23) - jnp.int32(127)
    mb = (bits & jnp.int32(0x007FFFFF)) | jnp.int32(0x3F800000)
    m = jax.lax.bitcast_convert_type(mb, jnp.float32)
    big = m >= _SQRT2
    m = jnp.where(big, m * jnp.float32(0.5), m)
    e = jnp.where(big, e + jnp.int32(1), e)
    ef = e.astype(jnp.float32)
    t = (m - jnp.float32(1.0)) / (m + jnp.float32(1.0))
    t2 = t * t
    p = jnp.float32(1.0 / 7.0)
    p = p * t2 + jnp.float32(1.0 / 5.0)
    p = p * t2 + jnp.float32(1.0 / 3.0)
    p = p * t2 + jnp.float32(1.0)
    return ef * _LN2 + (t + t) * p


def _make_sc(B, V, vlo, vsc, CC, NS):
    nch = vsc // CC
    nvec = CC // 16
    mesh = plsc.VectorSubcoreMesh(core_axis_name="c", subcore_axis_name="s")

    @functools.partial(
        pl.kernel,
        mesh=mesh,
        out_type=(jax.ShapeDtypeStruct((B, 208), jnp.float32),
                  jax.ShapeDtypeStruct((B, 160), jnp.int32)),
        scratch_types=[pltpu.VMEM((CC,), jnp.float32),
                       pltpu.VMEM((208,), jnp.float32),
                       pltpu.VMEM((160,), jnp.int32)],
    )
    def sc_kern(l_hbm, outf_hbm, outi_hbm, buf, obf, obi):
        w = lax.axis_index("s") * 2 + lax.axis_index("c")
        lane = lax.iota(jnp.int32, 16)

        m0 = jnp.full((16,), -3e38, jnp.float32)
        s0 = jnp.zeros((16,), jnp.float32)
        q0 = tuple(jnp.full((16,), 3e38, jnp.float32) for _ in range(NS))
        bc0 = tuple(jnp.zeros((16,), jnp.int32) for _ in range(NS))

        def chunk(c, carry):
            pltpu.sync_copy(l_hbm.at[pl.ds(w * vsc + c * CC, CC)], buf)
            base = vlo + c * CC

            def vec(t, carry):
                m, ssum, qs, bcs = carry
                l = buf[pl.ds(t * 16, 16)]
                col = lane + (base + t * 16)
                m = jnp.maximum(m, l)
                ssum = ssum + jnp.exp(l)
                wneg = jnp.exp(-l)
                offu = jax.lax.bitcast_convert_type(col + w * V, jnp.uint32)
                nq = []
                nbc = []
                for s in range(NS):
                    u = _bits_to_u(_threefry_bits(offu + jnp.uint32(s * B * V)))
                    q = -_ln(u) * wneg
                    bt = q < qs[s]
                    nq.append(jnp.where(bt, q, qs[s]))
                    nbc.append(jnp.where(bt, col, bcs[s]))
                return (m, ssum, tuple(nq), tuple(nbc))

            return lax.fori_loop(0, nvec, vec, carry, unroll=2)

        m, ssum, qs, bcs = lax.fori_loop(0, nch, chunk, (m0, s0, q0, bc0))

        zf = jnp.zeros((16,), jnp.float32)
        for s in range(NS):
            obf[pl.ds(s * 16, 16)] = qs[s]
            obi[pl.ds(s * 16, 16)] = bcs[s]
        obf[pl.ds(NS * 16, 16)] = m
        obf[pl.ds(NS * 16 + 16, 16)] = ssum
        obf[pl.ds(NS * 16 + 32, 16)] = zf
        pltpu.sync_copy(obf, outf_hbm.at[w])
        pltpu.sync_copy(obi, outi_hbm.at[w])

    return sc_kern


def _make_merge(B, NS, interpret=False):
    def body(tf_ref, ti_ref, sf_ref, si_ref, out_ref):
        intmax = jnp.int32(0x7FFFFFFF)
        m_row = jnp.maximum(
            tf_ref[:, NS : NS + 1],
            jnp.max(sf_ref[:, NS * 16 : NS * 16 + 16], axis=1, keepdims=True))
        s_row = tf_ref[:, NS + 1 : NS + 2] + jnp.sum(
            sf_ref[:, NS * 16 + 16 : NS * 16 + 32], axis=1, keepdims=True)
        top_prob = jnp.exp(m_row) / s_row
        idxs = []
        for s in range(NS):
            sq = sf_ref[:, s * 16 : (s + 1) * 16]
            sminq = jnp.min(sq, axis=1, keepdims=True)
            scand = jnp.where(sq == sminq, si_ref[:, s * 16 : (s + 1) * 16], intmax)
            sidx = jnp.min(scand, axis=1, keepdims=True)
            use_sc = -jnp.log(sminq) > tf_ref[:, s : s + 1]
            idxs.append(jnp.where(use_sc, sidx, ti_ref[:, s : s + 1]))
        agree_f = jnp.ones_like(top_prob)
        for t in range(1, NS):
            agree_f = agree_f + (idxs[t] == idxs[0]).astype(jnp.float32)
        agree_f = agree_f * jnp.float32(1.0 / NS)
        uniq = jnp.ones_like(top_prob)
        for t in range(1, NS):
            seen = idxs[t] == idxs[0]
            for t2 in range(1, t):
                seen = jnp.logical_or(seen, idxs[t] == idxs[t2])
            uniq = uniq + jnp.float32(1.0) - seen.astype(jnp.float32)
        out_ref[...] = jnp.zeros_like(out_ref)
        out_ref[:, 0:1] = agree_f
        out_ref[:, 1:2] = uniq * jnp.float32(1.0 / NS)
        out_ref[:, 2:3] = agree_f - top_prob

    return pl.pallas_call(
        body,
        out_shape=jax.ShapeDtypeStruct((B, 128), jnp.float32),
        interpret=interpret,
    )


def kernel(logits):
    lsc = logits[:, _VTC:].reshape(_B * _VSC)
    tcf, tci = _make_tc(_B, _V, _VTC, _CHUNK, _NS)(logits)
    scf, sci = _make_sc(_B, _V, _VTC, _VSC, _CC, _NS)(lsc)
    out = _make_merge(_B, _NS)(tcf, tci, scf, sci)
    return out[:, :3]
